# baseline (device time: 15228 ns/iter reference)
import jax
import jax.numpy as jnp
from jax import lax
from jax.experimental import pallas as pl
from jax.experimental.pallas import tpu as pltpu

N_DEV = 4
S = 4


def _gelu(z):
    return 0.5 * z * (1.0 + jnp.tanh(0.7978845608 * (z + 0.044715 * z * z * z)))


def kernel(A, B):
    m, k = A.shape
    k2, n = B.shape
    mc = m // N_DEV
    mcs = mc // S

    def body(a_hbm, b_hbm, out_hbm, avm, bvm, pbuf, rs_buf, gbuf, ag_buf,
             in_sems, out_sems,
             rs_send_sems, rs_recv_sems, ag_send_sems, ag_recv_sems):
        my_pos = lax.axis_index("i")

        copy_a = pltpu.make_async_copy(a_hbm, avm, in_sems.at[0])
        copy_b = pltpu.make_async_copy(b_hbm, bvm, in_sems.at[1])
        copy_a.start()
        copy_b.start()

        barrier_sem = pltpu.get_barrier_semaphore()
        for o in range(1, N_DEV):
            pl.semaphore_signal(
                barrier_sem, inc=1,
                device_id=((my_pos + o) % N_DEV,),
                device_id_type=pl.DeviceIdType.MESH,
            )

        copy_b.wait()
        copy_a.wait()
        b_bf16 = bvm[:, :].astype(jnp.bfloat16)

        rs_sends = []
        for o in range(1, N_DEV):
            dest = (my_pos + o) % N_DEV
            pc = jnp.dot(
                avm[pl.ds(dest * mc, mc), :].astype(jnp.bfloat16),
                b_bf16,
                preferred_element_type=jnp.float32,
            )
            pbuf[dest] = pc.astype(jnp.bfloat16).reshape(S, mcs, n)
            if o == 1:
                pl.semaphore_wait(barrier_sem, N_DEV - 1)
            for s in range(S):
                rdma = pltpu.make_async_remote_copy(
                    src_ref=pbuf.at[dest, s],
                    dst_ref=rs_buf.at[my_pos, s],
                    send_sem=rs_send_sems.at[o - 1, s],
                    recv_sem=rs_recv_sems.at[my_pos, s],
                    device_id=(dest,),
                    device_id_type=pl.DeviceIdType.MESH,
                )
                rdma.start()
                rs_sends.append(rdma)

        own = jnp.dot(
            avm[pl.ds(my_pos * mc, mc), :].astype(jnp.bfloat16),
            b_bf16,
            preferred_element_type=jnp.float32,
        ).reshape(S, mcs, n)

        ag_sends = []
        out_copies = []
        for s in range(S):
            z = own[s]
            for o in range(1, N_DEV):
                src = (my_pos - o) % N_DEV
                recv = pltpu.make_async_remote_copy(
                    src_ref=pbuf.at[0, 0],
                    dst_ref=rs_buf.at[src, s],
                    send_sem=rs_send_sems.at[o - 1, s],
                    recv_sem=rs_recv_sems.at[src, s],
                    device_id=(src,),
                    device_id_type=pl.DeviceIdType.MESH,
                )
                recv.wait_recv()
                z = z + rs_buf[src, s, :, :].astype(jnp.float32)
            g = _gelu(z).astype(jnp.bfloat16)
            gbuf[s, :, :] = g
            for o in range(1, N_DEV):
                dest = (my_pos + o) % N_DEV
                rdma = pltpu.make_async_remote_copy(
                    src_ref=gbuf.at[s],
                    dst_ref=ag_buf.at[my_pos, s],
                    send_sem=ag_send_sems.at[o - 1, s],
                    recv_sem=ag_recv_sems.at[my_pos, s],
                    device_id=(dest,),
                    device_id_type=pl.DeviceIdType.MESH,
                )
                rdma.start()
                ag_sends.append(rdma)
            cp = pltpu.make_async_copy(
                gbuf.at[s],
                out_hbm.at[pl.ds(my_pos * mc + s * mcs, mcs), :],
                out_sems.at[my_pos, s],
            )
            cp.start()
            out_copies.append(cp)

        for s in range(S):
            for o in range(1, N_DEV):
                src = (my_pos - o) % N_DEV
                recv = pltpu.make_async_remote_copy(
                    src_ref=gbuf.at[0],
                    dst_ref=ag_buf.at[src, s],
                    send_sem=ag_send_sems.at[o - 1, s],
                    recv_sem=ag_recv_sems.at[src, s],
                    device_id=(src,),
                    device_id_type=pl.DeviceIdType.MESH,
                )
                recv.wait_recv()
                cp = pltpu.make_async_copy(
                    ag_buf.at[src, s],
                    out_hbm.at[pl.ds(src * mc + s * mcs, mcs), :],
                    out_sems.at[src, s],
                )
                cp.start()
                out_copies.append(cp)

        for cp in out_copies:
            cp.wait()
        for rdma in rs_sends + ag_sends:
            rdma.wait_send()

    return pl.pallas_call(
        body,
        out_shape=jax.ShapeDtypeStruct((m, n), jnp.bfloat16),
        in_specs=[
            pl.BlockSpec(memory_space=pl.ANY),
            pl.BlockSpec(memory_space=pl.ANY),
        ],
        out_specs=pl.BlockSpec(memory_space=pl.ANY),
        scratch_shapes=[
            pltpu.VMEM((m, k), jnp.float32),
            pltpu.VMEM((k, n), jnp.float32),
            pltpu.VMEM((N_DEV, S, mcs, n), jnp.bfloat16),
            pltpu.VMEM((N_DEV, S, mcs, n), jnp.bfloat16),
            pltpu.VMEM((S, mcs, n), jnp.bfloat16),
            pltpu.VMEM((N_DEV, S, mcs, n), jnp.bfloat16),
            pltpu.SemaphoreType.DMA((2,)),
            pltpu.SemaphoreType.DMA((N_DEV, S)),
            pltpu.SemaphoreType.DMA((N_DEV - 1, S)),
            pltpu.SemaphoreType.DMA((N_DEV, S)),
            pltpu.SemaphoreType.DMA((N_DEV - 1, S)),
            pltpu.SemaphoreType.DMA((N_DEV, S)),
        ],
        compiler_params=pltpu.CompilerParams(collective_id=0),
    )(A, B)


# device time: 14483 ns/iter; 1.0514x vs baseline; 1.0514x over previous
import jax
import jax.numpy as jnp
from jax import lax
from jax.experimental import pallas as pl
from jax.experimental.pallas import tpu as pltpu

N_DEV = 4
S = 4


def _gelu(z):
    return 0.5 * z * (1.0 + jnp.tanh(0.7978845608 * (z + 0.044715 * z * z * z)))


def kernel(A, B):
    m, k = A.shape
    k2, n = B.shape
    mc = m // N_DEV
    mcs = mc // S

    def body(a_ref, b_ref, out_ref, pbuf, rs_buf, gbuf, ag_buf,
             rs_send_sems, rs_recv_sems, ag_send_sems, ag_recv_sems):
        my_pos = lax.axis_index("i")

        barrier_sem = pltpu.get_barrier_semaphore()
        for o in range(1, N_DEV):
            pl.semaphore_signal(
                barrier_sem, inc=1,
                device_id=((my_pos + o) % N_DEV,),
                device_id_type=pl.DeviceIdType.MESH,
            )

        b_bf16 = b_ref[:, :].astype(jnp.bfloat16)

        rs_sends = []
        for o in range(1, N_DEV):
            dest = (my_pos + o) % N_DEV
            pc = jnp.dot(
                a_ref[pl.ds(dest * mc, mc), :].astype(jnp.bfloat16),
                b_bf16,
                preferred_element_type=jnp.float32,
            )
            pbuf[dest] = pc.astype(jnp.bfloat16).reshape(S, mcs, n)
            if o == 1:
                pl.semaphore_wait(barrier_sem, N_DEV - 1)
            for s in range(S):
                rdma = pltpu.make_async_remote_copy(
                    src_ref=pbuf.at[dest, s],
                    dst_ref=rs_buf.at[my_pos, s],
                    send_sem=rs_send_sems.at[o - 1, s],
                    recv_sem=rs_recv_sems.at[my_pos, s],
                    device_id=(dest,),
                    device_id_type=pl.DeviceIdType.MESH,
                )
                rdma.start()
                rs_sends.append(rdma)

        own = jnp.dot(
            a_ref[pl.ds(my_pos * mc, mc), :].astype(jnp.bfloat16),
            b_bf16,
            preferred_element_type=jnp.float32,
        ).reshape(S, mcs, n)

        ag_sends = []
        for s in range(S):
            z = own[s]
            for o in range(1, N_DEV):
                src = (my_pos - o) % N_DEV
                recv = pltpu.make_async_remote_copy(
                    src_ref=pbuf.at[0, 0],
                    dst_ref=rs_buf.at[src, s],
                    send_sem=rs_send_sems.at[o - 1, s],
                    recv_sem=rs_recv_sems.at[src, s],
                    device_id=(src,),
                    device_id_type=pl.DeviceIdType.MESH,
                )
                recv.wait_recv()
                z = z + rs_buf[src, s, :, :].astype(jnp.float32)
            g = _gelu(z).astype(jnp.bfloat16)
            gbuf[s, :, :] = g
            for o in range(1, N_DEV):
                dest = (my_pos + o) % N_DEV
                rdma = pltpu.make_async_remote_copy(
                    src_ref=gbuf.at[s],
                    dst_ref=ag_buf.at[my_pos, s],
                    send_sem=ag_send_sems.at[o - 1, s],
                    recv_sem=ag_recv_sems.at[my_pos, s],
                    device_id=(dest,),
                    device_id_type=pl.DeviceIdType.MESH,
                )
                rdma.start()
                ag_sends.append(rdma)
            out_ref[pl.ds(my_pos * mc + s * mcs, mcs), :] = g

        for s in range(S):
            for o in range(1, N_DEV):
                src = (my_pos - o) % N_DEV
                recv = pltpu.make_async_remote_copy(
                    src_ref=gbuf.at[0],
                    dst_ref=ag_buf.at[src, s],
                    send_sem=ag_send_sems.at[o - 1, s],
                    recv_sem=ag_recv_sems.at[src, s],
                    device_id=(src,),
                    device_id_type=pl.DeviceIdType.MESH,
                )
                recv.wait_recv()
                out_ref[pl.ds(src * mc + s * mcs, mcs), :] = ag_buf[src, s, :, :]

        for rdma in rs_sends + ag_sends:
            rdma.wait_send()

    return pl.pallas_call(
        body,
        out_shape=jax.ShapeDtypeStruct((m, n), jnp.bfloat16),
        in_specs=[
            pl.BlockSpec(memory_space=pltpu.VMEM),
            pl.BlockSpec(memory_space=pltpu.VMEM),
        ],
        out_specs=pl.BlockSpec(memory_space=pltpu.VMEM),
        scratch_shapes=[
            pltpu.VMEM((N_DEV, S, mcs, n), jnp.bfloat16),
            pltpu.VMEM((N_DEV, S, mcs, n), jnp.bfloat16),
            pltpu.VMEM((S, mcs, n), jnp.bfloat16),
            pltpu.VMEM((N_DEV, S, mcs, n), jnp.bfloat16),
            pltpu.SemaphoreType.DMA((N_DEV - 1, S)),
            pltpu.SemaphoreType.DMA((N_DEV, S)),
            pltpu.SemaphoreType.DMA((N_DEV - 1, S)),
            pltpu.SemaphoreType.DMA((N_DEV, S)),
        ],
        compiler_params=pltpu.CompilerParams(collective_id=0),
    )(A, B)
